# baseline (speedup 1.0000x reference)
import functools

import jax
import jax.numpy as jnp
from jax import lax
from jax.experimental import pallas as pl
from jax.experimental.pallas import tpu as pltpu
from jax.experimental.pallas import tpu_sc as plsc

_R = 32
_NSLOT = 64
_L = 16


def _sc_bybt_body(M, B, idx_hbm, y_hbm, by_hbm, bt_hbm, t_hbm,
                  by_out, bt_out,
                  idx_v, y_v, by_v, bt_v, t_v, srt_v):
    wid = lax.axis_index("s") * 2 + lax.axis_index("c")

    @pl.when(wid == 0)
    def _():
        pltpu.sync_copy(idx_hbm, idx_v)
        pltpu.sync_copy(y_hbm, y_v)
        pltpu.sync_copy(by_hbm, by_v)
        pltpu.sync_copy(bt_hbm, bt_v)
        pltpu.sync_copy(t_hbm, t_v)
        tvec = t_v[...]
        io = lax.broadcasted_iota(jnp.int32, (_L,), 0)

        def _chunk(c, _):
            j0 = c * _L
            iv = idx_v[pl.ds(j0, _L)]
            valid = iv < M
            ivc = jnp.where(valid, iv, M - 1)
            comp = ivc * 512 + io
            sk, sl = plsc.sort_key_val(comp, io)
            srt_v[...] = sk
            nxt = plsc.load_gather(srt_v, [jnp.minimum(io + 1, _L - 1)])
            last = jnp.logical_or((nxt >> 9) != (sk >> 9), io == _L - 1)
            plsc.store_scatter(srt_v, [sl], jnp.where(last, 1, 0))
            win = jnp.logical_and(srt_v[...] != 0, valid)
            plsc.store_scatter(by_v, [ivc], y_v[pl.ds(j0, _L)], mask=win)
            plsc.store_scatter(bt_v, [ivc], tvec, mask=win)
            return 0
        jax.lax.fori_loop(0, B // _L, _chunk, 0)

        pltpu.sync_copy(by_v, by_out)
        pltpu.sync_copy(bt_v, bt_out)


def _tc_dense_body(M, B, idx_smem, x_any, bx_ref, out_ref,
                   wjrow_smem, blkmask_smem, ctr_smem, buf, sems):
    i = pl.program_id(0)
    nblk = M // _R

    def _issue_block(blk_id):
        mask = blkmask_smem[blk_id]
        for k in range(_R):
            @pl.when(((mask >> k) & 1) != 0)
            def _():
                m = blk_id * _R + k
                q = ctr_smem[0]
                pltpu.make_async_copy(
                    x_any.at[wjrow_smem[m]], buf.at[q % _NSLOT],
                    sems.at[q % _NSLOT]).start()
                ctr_smem[0] = q + 1

    @pl.when(i == 0)
    def _meta():
        def _z(b, _):
            blkmask_smem[b] = 0
            return 0
        jax.lax.fori_loop(0, nblk, _z, 0)
        ctr_smem[0] = 0
        ctr_smem[1] = 0

        def _scan(jr, _):
            j = B - 1 - jr
            iv = idx_smem[j]
            ivc = jnp.minimum(iv, M - 1)
            b = ivc // _R
            bit = jnp.int32(1) << (ivc % _R)

            @pl.when(jnp.logical_and(iv < M, (blkmask_smem[b] & bit) == 0))
            def _():
                blkmask_smem[b] = blkmask_smem[b] | bit
                wjrow_smem[ivc] = j
            return 0
        jax.lax.fori_loop(0, B, _scan, 0)

        _issue_block(0)

    @pl.when(i + 1 < nblk)
    def _issue_next():
        _issue_block(i + 1)

    out_ref[...] = bx_ref[...]

    mask = blkmask_smem[i]
    for k in range(_R):
        @pl.when(((mask >> k) & 1) != 0)
        def _():
            p = ctr_smem[1]
            slot = p % _NSLOT
            pltpu.make_async_copy(
                x_any.at[wjrow_smem[i * _R + k]], buf.at[slot],
                sems.at[slot]).wait()
            out_ref[pl.ds(k, 1)] = buf[pl.ds(slot, 1)]
            ctr_smem[1] = p + 1


def kernel(bx, by, bt, x, y, indices, t):
    M = bx.shape[0]
    B = x.shape[0]
    row_shape = bx.shape[1:]
    nblk = M // _R
    t_arr = jnp.full((_L,), t, jnp.int32)

    mesh = plsc.VectorSubcoreMesh(core_axis_name="c", subcore_axis_name="s")
    sc_bybt = pl.kernel(
        functools.partial(_sc_bybt_body, M, B),
        out_type=[
            jax.ShapeDtypeStruct(by.shape, by.dtype),
            jax.ShapeDtypeStruct(bt.shape, bt.dtype),
        ],
        mesh=mesh,
        compiler_params=pltpu.CompilerParams(needs_layout_passes=False),
        scratch_types=[
            pltpu.VMEM((B,), jnp.int32),
            pltpu.VMEM((B,), jnp.int32),
            pltpu.VMEM((M,), jnp.int32),
            pltpu.VMEM((M,), jnp.int32),
            pltpu.VMEM((_L,), jnp.int32),
            pltpu.VMEM((_L,), jnp.int32),
        ],
    )
    new_by, new_bt = sc_bybt(indices, y, by, bt, t_arr)

    smem = functools.partial(pl.BlockSpec, memory_space=pltpu.SMEM)
    anys = functools.partial(pl.BlockSpec, memory_space=pltpu.MemorySpace.HBM)
    blk = (_R,) + row_shape
    zeros = (0,) * len(row_shape)

    new_bx = pl.pallas_call(
        functools.partial(_tc_dense_body, M, B),
        grid=(nblk,),
        in_specs=[smem(), anys(),
                  pl.BlockSpec(blk, lambda i: (i,) + zeros)],
        out_specs=pl.BlockSpec(blk, lambda i: (i,) + zeros),
        out_shape=jax.ShapeDtypeStruct(bx.shape, bx.dtype),
        scratch_shapes=[
            pltpu.SMEM((M,), jnp.int32),
            pltpu.SMEM((nblk,), jnp.int32),
            pltpu.SMEM((2,), jnp.int32),
            pltpu.VMEM((_NSLOT,) + row_shape, bx.dtype),
            pltpu.SemaphoreType.DMA((_NSLOT,)),
        ],
    )(indices, x, bx)
    return (new_bx, new_by, new_bt)


# speedup vs baseline: 1.0432x; 1.0432x over previous
import functools

import jax
import jax.numpy as jnp
from jax import lax
from jax.experimental import pallas as pl
from jax.experimental.pallas import tpu as pltpu
from jax.experimental.pallas import tpu_sc as plsc

_R = 32
_NSLOT = 64
_L = 16


def _sc_meta_body(M, B, idx_hbm, y_hbm, by_hbm, bt_hbm, t_hbm,
                  by_out, bt_out, wjrow_out,
                  idx_v, y_v, by_v, bt_v, t_v, wj_v, srt_v,
                  s0, s1, s2, s3, s4):
    wid = lax.axis_index("s") * 2 + lax.axis_index("c")

    @pl.when(wid == 0)
    def _():
        h0 = pltpu.async_copy(idx_hbm, idx_v, s0)
        h1 = pltpu.async_copy(y_hbm, y_v, s1)
        h2 = pltpu.async_copy(by_hbm, by_v, s2)
        h3 = pltpu.async_copy(bt_hbm, bt_v, s3)
        h4 = pltpu.async_copy(t_hbm, t_v, s4)
        h0.wait()
        h1.wait()
        h2.wait()
        h3.wait()
        h4.wait()
        tvec = t_v[...]
        io = lax.broadcasted_iota(jnp.int32, (_L,), 0)
        neg1 = jnp.full((_L,), -1, jnp.int32)

        def _init(c, _):
            wj_v[pl.ds(c * _L, _L)] = neg1
            return 0
        jax.lax.fori_loop(0, M // _L, _init, 0)

        def _chunk(c, _):
            j0 = c * _L
            iv = idx_v[pl.ds(j0, _L)]
            valid = iv < M
            ivc = jnp.where(valid, iv, M - 1)
            comp = ivc * 512 + io
            sk, sl = plsc.sort_key_val(comp, io)
            srt_v[...] = sk
            nxt = plsc.load_gather(srt_v, [jnp.minimum(io + 1, _L - 1)])
            last = jnp.logical_or((nxt >> 9) != (sk >> 9), io == _L - 1)
            plsc.store_scatter(srt_v, [sl], jnp.where(last, 1, 0))
            win = jnp.logical_and(srt_v[...] != 0, valid)
            plsc.store_scatter(wj_v, [ivc], j0 + io, mask=win)
            plsc.store_scatter(by_v, [ivc], y_v[pl.ds(j0, _L)], mask=win)
            plsc.store_scatter(bt_v, [ivc], tvec, mask=win)
            return 0
        jax.lax.fori_loop(0, B // _L, _chunk, 0)

        g0 = pltpu.async_copy(by_v, by_out, s0)
        g1 = pltpu.async_copy(bt_v, bt_out, s1)
        g2 = pltpu.async_copy(wj_v, wjrow_out, s2)
        g0.wait()
        g1.wait()
        g2.wait()


def _tc_dense_body(M, wjrow_smem, x_any, bx_ref, out_ref,
                   ctr_smem, buf, sems):
    i = pl.program_id(0)
    nblk = M // _R

    def _issue_block(blk_id):
        for k in range(_R):
            m = blk_id * _R + k
            w = wjrow_smem[m]

            @pl.when(w >= 0)
            def _():
                q = ctr_smem[0]
                pltpu.make_async_copy(
                    x_any.at[w], buf.at[q % _NSLOT],
                    sems.at[q % _NSLOT]).start()
                ctr_smem[0] = q + 1

    @pl.when(i == 0)
    def _():
        ctr_smem[0] = 0
        ctr_smem[1] = 0
        _issue_block(0)

    @pl.when(i + 1 < nblk)
    def _():
        _issue_block(i + 1)

    out_ref[...] = bx_ref[...]

    for k in range(_R):
        w = wjrow_smem[i * _R + k]

        @pl.when(w >= 0)
        def _():
            p = ctr_smem[1]
            slot = p % _NSLOT
            pltpu.make_async_copy(
                x_any.at[w], buf.at[slot], sems.at[slot]).wait()
            out_ref[pl.ds(k, 1)] = buf[pl.ds(slot, 1)]
            ctr_smem[1] = p + 1


def kernel(bx, by, bt, x, y, indices, t):
    M = bx.shape[0]
    B = x.shape[0]
    row_shape = bx.shape[1:]
    nblk = M // _R
    t_arr = jnp.full((_L,), t, jnp.int32)

    mesh = plsc.VectorSubcoreMesh(core_axis_name="c", subcore_axis_name="s")
    sc_meta = pl.kernel(
        functools.partial(_sc_meta_body, M, B),
        out_type=[
            jax.ShapeDtypeStruct(by.shape, by.dtype),
            jax.ShapeDtypeStruct(bt.shape, bt.dtype),
            jax.ShapeDtypeStruct((M,), jnp.int32),
        ],
        mesh=mesh,
        compiler_params=pltpu.CompilerParams(needs_layout_passes=False),
        scratch_types=[
            pltpu.VMEM((B,), jnp.int32),
            pltpu.VMEM((B,), jnp.int32),
            pltpu.VMEM((M,), jnp.int32),
            pltpu.VMEM((M,), jnp.int32),
            pltpu.VMEM((_L,), jnp.int32),
            pltpu.VMEM((M,), jnp.int32),
            pltpu.VMEM((_L,), jnp.int32),
            pltpu.SemaphoreType.DMA,
            pltpu.SemaphoreType.DMA,
            pltpu.SemaphoreType.DMA,
            pltpu.SemaphoreType.DMA,
            pltpu.SemaphoreType.DMA,
        ],
    )
    new_by, new_bt, wjrow = sc_meta(indices, y, by, bt, t_arr)

    smem = functools.partial(pl.BlockSpec, memory_space=pltpu.SMEM)
    anys = functools.partial(pl.BlockSpec, memory_space=pltpu.MemorySpace.HBM)
    blk = (_R,) + row_shape
    zeros = (0,) * len(row_shape)

    new_bx = pl.pallas_call(
        functools.partial(_tc_dense_body, M),
        grid=(nblk,),
        in_specs=[smem(), anys(),
                  pl.BlockSpec(blk, lambda i: (i,) + zeros)],
        out_specs=pl.BlockSpec(blk, lambda i: (i,) + zeros),
        out_shape=jax.ShapeDtypeStruct(bx.shape, bx.dtype),
        scratch_shapes=[
            pltpu.SMEM((2,), jnp.int32),
            pltpu.VMEM((_NSLOT,) + row_shape, bx.dtype),
            pltpu.SemaphoreType.DMA((_NSLOT,)),
        ],
    )(wjrow, x, bx)
    return (new_bx, new_by, new_bt)
